# baseline (speedup 1.0000x reference)
import jax
import jax.numpy as jnp
from jax.experimental import pallas as pl

B, N, DIM = 2, 4096, 256
H, DH, K = 8, 64, 16
INNER = H * DH
SCALE = DH ** -0.5


def kernel(x, center_pos3d, Wq, Wkv, Wout, bout):
    pos = center_pos3d
    d2 = jnp.sum(pos * pos, axis=-1)
    dist = d2[:, :, None] + d2[:, None, :] - 2.0 * jnp.einsum('bic,bjc->bij', pos, pos)
    _, idx = jax.lax.top_k(-dist, K)
    knn_x = jax.vmap(lambda xb, ib: xb[ib])(x, idx)
    q = x @ Wq.T
    kv = knn_x @ Wkv.T
    k_, v_ = jnp.split(kv, 2, axis=-1)
    q = q.reshape(B, N, H, DH).transpose(0, 2, 1, 3)[:, :, :, None, :]
    k_ = k_.reshape(B, N, K, H, DH).transpose(0, 3, 1, 2, 4)
    v_ = v_.reshape(B, N, K, H, DH).transpose(0, 3, 1, 2, 4)
    dots = jnp.einsum('bhnid,bhnjd->bhnij', q, k_) * SCALE
    attn = jax.nn.softmax(dots, axis=-1)
    out = jnp.einsum('bhnij,bhnjd->bhnid', attn, v_)
    out = out[:, :, :, 0, :].transpose(0, 2, 1, 3).reshape(B, N, INNER)
    return out @ Wout.T + bout


# speedup vs baseline: 7.6351x; 7.6351x over previous
import functools

import jax
import jax.numpy as jnp
from jax import lax
from jax.experimental import pallas as pl
from jax.experimental.pallas import tpu as pltpu
from jax.experimental.pallas import tpu_sc as plsc

B, N, DIM = 2, 4096, 256
H, DH, K = 8, 64, 16
INNER = H * DH
SCALE = DH ** -0.5

RBLK = 256
PBLK = 256
_BIG = 3.0e38


def _knn_body(pa_ref, pb_ref, idx_ref, d_ref):
    b = pl.program_id(0)
    g = lax.dot_general(pa_ref[0], pb_ref[0], (((1,), (0,)), ((), ())),
                        preferred_element_type=jnp.float32)
    d2col = pa_ref[0][:, 3:4]
    d2row = pb_ref[0][4:5, :]
    d_ref[...] = (d2col + d2row) - 2.0 * g
    iota = lax.broadcasted_iota(jnp.int32, (RBLK, N), 1)
    cols = []
    for _ in range(K):
        d = d_ref[...]
        m = jnp.min(d, axis=1, keepdims=True)
        cand = jnp.where(d <= m, iota, jnp.int32(N))
        a = jnp.min(cand, axis=1, keepdims=True)
        cols.append(a)
        d_ref[...] = jnp.where(iota == a, _BIG, d)
    idx_ref[0] = jnp.concatenate(cols, axis=1) + b * N


def _knn_idx(pa, pb):
    return pl.pallas_call(
        _knn_body,
        grid=(B, N // RBLK),
        in_specs=[
            pl.BlockSpec((1, RBLK, 8), lambda b, i: (b, i, 0)),
            pl.BlockSpec((1, 8, N), lambda b, i: (b, 0, 0)),
        ],
        out_specs=pl.BlockSpec((1, RBLK, K), lambda b, i: (b, i, 0)),
        out_shape=jax.ShapeDtypeStruct((B, N, K), jnp.int32),
        scratch_shapes=[pltpu.VMEM((RBLK, N), jnp.float32)],
    )(pa, pb)


_SC_WORKERS = 32
_GCHUNK = 256


def _gather_sc(table, idxflat):
    rows_total = idxflat.shape[0]
    per_w = rows_total // _SC_WORKERS
    n_chunks = per_w // _GCHUNK
    mesh = plsc.VectorSubcoreMesh(core_axis_name="c", subcore_axis_name="s")

    @functools.partial(
        pl.kernel, mesh=mesh,
        out_type=jax.ShapeDtypeStruct((rows_total, DIM), jnp.float32),
        scratch_types=[
            pltpu.VMEM((_GCHUNK,), jnp.int32),
            pltpu.VMEM((_GCHUNK, DIM), jnp.float32),
            pltpu.SemaphoreType.DMA,
        ],
    )
    def k(table_hbm, idx_hbm, out_hbm, idx_v, rows_v, sem):
        wid = lax.axis_index("s") * 2 + lax.axis_index("c")
        for t in range(n_chunks):
            base = wid * per_w + t * _GCHUNK
            pltpu.sync_copy(idx_hbm.at[pl.ds(base, _GCHUNK)], idx_v)
            pltpu.async_copy(table_hbm.at[idx_v], rows_v, sem).wait()
            pltpu.sync_copy(rows_v, out_hbm.at[pl.ds(base, _GCHUNK)])

    return k(table, idxflat)


def _attn_body(x_ref, g_ref, wq_ref, wkv_ref, wo_ref, bo_ref, y_ref):
    q = lax.dot_general(x_ref[...], wq_ref[...], (((1,), (1,)), ((), ())),
                        preferred_element_type=jnp.float32)
    kv = lax.dot_general(g_ref[...], wkv_ref[...], (((1,), (1,)), ((), ())),
                         preferred_element_type=jnp.float32)
    kk = kv[:, :INNER].reshape(PBLK, K, INNER)
    vv = kv[:, INNER:].reshape(PBLK, K, INNER)
    outs = []
    for h in range(H):
        sl = slice(h * DH, (h + 1) * DH)
        qh = q[:, sl]
        kh = kk[:, :, sl]
        vh = vv[:, :, sl]
        dots = jnp.sum(qh[:, None, :] * kh, axis=-1) * SCALE
        m = jnp.max(dots, axis=-1, keepdims=True)
        e = jnp.exp(dots - m)
        a = e / jnp.sum(e, axis=-1, keepdims=True)
        outs.append(jnp.sum(a[:, :, None] * vh, axis=1))
    o = jnp.concatenate(outs, axis=1)
    y = lax.dot_general(o, wo_ref[...], (((1,), (1,)), ((), ())),
                        preferred_element_type=jnp.float32)
    y_ref[...] = y + bo_ref[...]


def _attention(x2, gx, Wq, Wkv, Wout, bout2):
    rows = x2.shape[0]
    return pl.pallas_call(
        _attn_body,
        grid=(rows // PBLK,),
        in_specs=[
            pl.BlockSpec((PBLK, DIM), lambda i: (i, 0)),
            pl.BlockSpec((PBLK * K, DIM), lambda i: (i, 0)),
            pl.BlockSpec((INNER, DIM), lambda i: (0, 0)),
            pl.BlockSpec((2 * INNER, DIM), lambda i: (0, 0)),
            pl.BlockSpec((DIM, INNER), lambda i: (0, 0)),
            pl.BlockSpec((1, DIM), lambda i: (0, 0)),
        ],
        out_specs=pl.BlockSpec((PBLK, DIM), lambda i: (i, 0)),
        out_shape=jax.ShapeDtypeStruct((rows, DIM), jnp.float32),
    )(x2, gx, Wq, Wkv, Wout, bout2)


def kernel(x, center_pos3d, Wq, Wkv, Wout, bout):
    pos = center_pos3d
    d2 = jnp.sum(pos * pos, axis=-1)
    zero1 = jnp.zeros((B, N, 1), jnp.float32)
    zero3 = jnp.zeros((B, N, 3), jnp.float32)
    pa = jnp.concatenate([pos, d2[..., None], zero1, zero3], axis=-1)
    pbr = jnp.concatenate([pos, zero1, d2[..., None], zero3], axis=-1)
    pb = pbr.transpose(0, 2, 1)

    idx = _knn_idx(pa, pb)
    x2 = x.reshape(B * N, DIM)
    gx = _gather_sc(x2, idx.reshape(B * N * K))
    y = _attention(x2, gx, Wq, Wkv, Wout, bout.reshape(1, DIM))
    return y.reshape(B, N, DIM)


# speedup vs baseline: 8.1364x; 1.0657x over previous
import functools

import jax
import jax.numpy as jnp
from jax import lax
from jax.experimental import pallas as pl
from jax.experimental.pallas import tpu as pltpu
from jax.experimental.pallas import tpu_sc as plsc

B, N, DIM = 2, 4096, 256
H, DH, K = 8, 64, 16
INNER = H * DH
SCALE = DH ** -0.5

RBLK = 256
PBLK = 256
_BIG = 3.0e38


def _knn_body(pa_ref, pb_ref, idx_ref, d_ref):
    b = pl.program_id(0)
    g = lax.dot_general(pa_ref[0], pb_ref[0], (((1,), (0,)), ((), ())),
                        preferred_element_type=jnp.float32)
    d2col = pa_ref[0][:, 3:4]
    d2row = pb_ref[0][4:5, :]
    d_ref[...] = (d2col + d2row) - 2.0 * g
    iota = lax.broadcasted_iota(jnp.int32, (RBLK, N), 1)
    cols = []
    for _ in range(K):
        d = d_ref[...]
        a = jnp.argmin(d, axis=1, keepdims=True).astype(jnp.int32)
        cols.append(a)
        d_ref[...] = jnp.where(iota == a, _BIG, d)
    idx_ref[0] = jnp.concatenate(cols, axis=1) + b * N


def _knn_idx(pa, pb):
    return pl.pallas_call(
        _knn_body,
        grid=(B, N // RBLK),
        in_specs=[
            pl.BlockSpec((1, RBLK, 8), lambda b, i: (b, i, 0)),
            pl.BlockSpec((1, 8, N), lambda b, i: (b, 0, 0)),
        ],
        out_specs=pl.BlockSpec((1, RBLK, K), lambda b, i: (b, i, 0)),
        out_shape=jax.ShapeDtypeStruct((B, N, K), jnp.int32),
        scratch_shapes=[pltpu.VMEM((RBLK, N), jnp.float32)],
    )(pa, pb)


_SC_WORKERS = 32
_GCHUNK = 256


def _gather_sc(table, idxflat):
    rows_total = idxflat.shape[0]
    per_w = rows_total // _SC_WORKERS
    n_chunks = per_w // _GCHUNK
    mesh = plsc.VectorSubcoreMesh(core_axis_name="c", subcore_axis_name="s")

    @functools.partial(
        pl.kernel, mesh=mesh,
        out_type=jax.ShapeDtypeStruct((rows_total, DIM), jnp.float32),
        scratch_types=[
            pltpu.VMEM((_GCHUNK,), jnp.int32),
            pltpu.VMEM((_GCHUNK, DIM), jnp.float32),
            pltpu.SemaphoreType.DMA,
        ],
    )
    def k(table_hbm, idx_hbm, out_hbm, idx_v, rows_v, sem):
        wid = lax.axis_index("s") * 2 + lax.axis_index("c")
        for t in range(n_chunks):
            base = wid * per_w + t * _GCHUNK
            pltpu.sync_copy(idx_hbm.at[pl.ds(base, _GCHUNK)], idx_v)
            pltpu.async_copy(table_hbm.at[idx_v], rows_v, sem).wait()
            pltpu.sync_copy(rows_v, out_hbm.at[pl.ds(base, _GCHUNK)])

    return k(table, idxflat)


def _attn_body(x_ref, g_ref, wq_ref, wkv_ref, wo_ref, bo_ref, y_ref):
    q = lax.dot_general(x_ref[...], wq_ref[...], (((1,), (1,)), ((), ())),
                        preferred_element_type=jnp.float32)
    kv = lax.dot_general(g_ref[...], wkv_ref[...], (((1,), (1,)), ((), ())),
                         preferred_element_type=jnp.float32)
    kk = kv[:, :INNER].reshape(PBLK, K, INNER)
    vv = kv[:, INNER:].reshape(PBLK, K, INNER)
    outs = []
    for h in range(H):
        sl = slice(h * DH, (h + 1) * DH)
        qh = q[:, sl]
        kh = kk[:, :, sl]
        vh = vv[:, :, sl]
        dots = jnp.sum(qh[:, None, :] * kh, axis=-1) * SCALE
        m = jnp.max(dots, axis=-1, keepdims=True)
        e = jnp.exp(dots - m)
        a = e / jnp.sum(e, axis=-1, keepdims=True)
        outs.append(jnp.sum(a[:, :, None] * vh, axis=1))
    o = jnp.concatenate(outs, axis=1)
    y = lax.dot_general(o, wo_ref[...], (((1,), (1,)), ((), ())),
                        preferred_element_type=jnp.float32)
    y_ref[...] = y + bo_ref[...]


def _attention(x2, gx, Wq, Wkv, Wout, bout2):
    rows = x2.shape[0]
    return pl.pallas_call(
        _attn_body,
        grid=(rows // PBLK,),
        in_specs=[
            pl.BlockSpec((PBLK, DIM), lambda i: (i, 0)),
            pl.BlockSpec((PBLK * K, DIM), lambda i: (i, 0)),
            pl.BlockSpec((INNER, DIM), lambda i: (0, 0)),
            pl.BlockSpec((2 * INNER, DIM), lambda i: (0, 0)),
            pl.BlockSpec((DIM, INNER), lambda i: (0, 0)),
            pl.BlockSpec((1, DIM), lambda i: (0, 0)),
        ],
        out_specs=pl.BlockSpec((PBLK, DIM), lambda i: (i, 0)),
        out_shape=jax.ShapeDtypeStruct((rows, DIM), jnp.float32),
    )(x2, gx, Wq, Wkv, Wout, bout2)


def kernel(x, center_pos3d, Wq, Wkv, Wout, bout):
    pos = center_pos3d
    d2 = jnp.sum(pos * pos, axis=-1)
    zero1 = jnp.zeros((B, N, 1), jnp.float32)
    zero3 = jnp.zeros((B, N, 3), jnp.float32)
    pa = jnp.concatenate([pos, d2[..., None], zero1, zero3], axis=-1)
    pbr = jnp.concatenate([pos, zero1, d2[..., None], zero3], axis=-1)
    pb = pbr.transpose(0, 2, 1)

    idx = _knn_idx(pa, pb)
    x2 = x.reshape(B * N, DIM)
    gx = _gather_sc(x2, idx.reshape(B * N * K))
    y = _attention(x2, gx, Wq, Wkv, Wout, bout.reshape(1, DIM))
    return y.reshape(B, N, DIM)
